# baseline (speedup 1.0000x reference)
import jax
import jax.numpy as jnp
from jax.experimental import pallas as pl
from jax.experimental.pallas import tpu as pltpu

_C = 100000
_IGNORE = 0
_SMOOTH = 0.1
_CONF = 1.0 - _SMOOTH
_EPS = _SMOOTH / (_C - 1)
_BV = 2048
_NSL = _BV // 128
_NBLK = (_C + _BV - 1) // _BV


def _body(tgt_ref, logit_ref, out_ref, acc_ref, accg_ref):
    j = pl.program_id(0)
    t = tgt_ref[...]
    tt = t - j * _BV
    col = jax.lax.broadcasted_iota(jnp.int32, (t.shape[0], 128), 1)

    @pl.when(j == 0)
    def _init():
        acc_ref[...] = jnp.zeros(acc_ref.shape, jnp.float32)
        accg_ref[...] = jnp.zeros(accg_ref.shape, jnp.float32)

    @pl.when(j < _NBLK - 1)
    def _full():
        a = acc_ref[...]
        g = accg_ref[...]
        for s in range(_NSL):
            sl = logit_ref[:, s * 128:(s + 1) * 128]
            a = a + sl
            g = g + jnp.where(col == tt - s * 128, sl, 0.0)
        acc_ref[...] = a
        accg_ref[...] = g

    @pl.when(j == _NBLK - 1)
    def _ragged():
        a = acc_ref[...]
        g = accg_ref[...]
        for s in range(_NSL):
            sl = logit_ref[:, s * 128:(s + 1) * 128]
            sl = jnp.where(col < _C - j * _BV - s * 128, sl, 0.0)
            a = a + sl
            g = g + jnp.where(col == tt - s * 128, sl, 0.0)
        row_ok = t != _IGNORE
        rowsum = jnp.sum(a, axis=1, keepdims=True)
        grow = jnp.sum(g, axis=1, keepdims=True)
        per_row = _EPS * rowsum + (_CONF - _EPS) * grow
        total = jnp.sum(jnp.where(row_ok, per_row, 0.0))
        out_ref[...] = jnp.full((1, 1), -total, jnp.float32)


def kernel(logit, target):
    n = logit.shape[0]
    tgt = target.astype(jnp.int32).reshape(n, 1)
    out = pl.pallas_call(
        _body,
        grid=(_NBLK,),
        in_specs=[
            pl.BlockSpec((n, 1), lambda j: (0, 0)),
            pl.BlockSpec((n, _BV), lambda j: (0, j)),
        ],
        out_specs=pl.BlockSpec((1, 1), lambda j: (0, 0)),
        out_shape=jax.ShapeDtypeStruct((1, 1), jnp.float32),
        scratch_shapes=[
            pltpu.VMEM((n, 128), jnp.float32),
            pltpu.VMEM((n, 128), jnp.float32),
        ],
        compiler_params=pltpu.CompilerParams(
            dimension_semantics=("arbitrary",),
        ),
    )(tgt, logit)
    return out[0, 0]


# speedup vs baseline: 1.1052x; 1.1052x over previous
import jax
import jax.numpy as jnp
from jax.experimental import pallas as pl
from jax.experimental.pallas import tpu as pltpu

_C = 100000
_IGNORE = 0
_SMOOTH = 0.1
_CONF = 1.0 - _SMOOTH
_EPS = _SMOOTH / (_C - 1)
_BV = 2048
_NBLK = (_C + _BV - 1) // _BV


def _body(tgt_ref, logit_ref, out_ref, accs_ref, accg_ref):
    j = pl.program_id(0)
    t = tgt_ref[...]
    tt = t - j * _BV
    r = t.shape[0]
    col = jax.lax.broadcasted_iota(jnp.int32, (r, _BV), 1)

    @pl.when(j == 0)
    def _init():
        accs_ref[...] = jnp.zeros(accs_ref.shape, jnp.float32)
        accg_ref[...] = jnp.zeros(accg_ref.shape, jnp.float32)

    @pl.when(j < _NBLK - 1)
    def _full():
        blk = logit_ref[...]
        accs_ref[...] += jnp.sum(blk, axis=1, keepdims=True)
        accg_ref[...] += jnp.sum(jnp.where(col == tt, blk, 0.0),
                                 axis=1, keepdims=True)

    @pl.when(j == _NBLK - 1)
    def _ragged():
        blk = jnp.where(col < _C - j * _BV, logit_ref[...], 0.0)
        s = accs_ref[...] + jnp.sum(blk, axis=1, keepdims=True)
        g = accg_ref[...] + jnp.sum(jnp.where(col == tt, blk, 0.0),
                                    axis=1, keepdims=True)
        per_row = _EPS * s + (_CONF - _EPS) * g
        total = jnp.sum(jnp.where(t != _IGNORE, per_row, 0.0))
        out_ref[...] = jnp.full((1, 1), -total, jnp.float32)


def kernel(logit, target):
    n = logit.shape[0]
    tgt = target.astype(jnp.int32).reshape(n, 1)
    out = pl.pallas_call(
        _body,
        grid=(_NBLK,),
        in_specs=[
            pl.BlockSpec((n, 1), lambda j: (0, 0)),
            pl.BlockSpec((n, _BV), lambda j: (0, j)),
        ],
        out_specs=pl.BlockSpec((1, 1), lambda j: (0, 0)),
        out_shape=jax.ShapeDtypeStruct((1, 1), jnp.float32),
        scratch_shapes=[
            pltpu.VMEM((n, 1), jnp.float32),
            pltpu.VMEM((n, 1), jnp.float32),
        ],
        compiler_params=pltpu.CompilerParams(
            dimension_semantics=("arbitrary",),
        ),
    )(tgt, logit)
    return out[0, 0]
